# baseline (speedup 1.0000x reference)
import functools

import jax
import jax.numpy as jnp
from jax import lax
from jax.experimental import pallas as pl
from jax.experimental.pallas import tpu as pltpu
from jax.experimental.pallas import tpu_sc as plsc

N = 10000
E = 320000
D = 128
NC, NS = 2, 16
NW = NC * NS
CHUNK = 128
NCHUNKS = 80
E_PAD = NW * NCHUNKS * CHUNK
N_PAD = 10240
ROWS_PER_TILE = N_PAD // NS
DEG_W = 16
B_OUT = 1024
B_PER_W = B_OUT // NW

_MESH = plsc.VectorSubcoreMesh(core_axis_name="c", subcore_axis_name="s",
                               num_cores=NC, num_subcores=NS)


def _sc_layer_body(with_deg, *refs):
    if with_deg:
        (src_r, dst_r, feats, zeros, zeros1, agg_out, deg_out,
         src_all, dst_all, rows_v, deg_v, agg_sh, sem_g) = refs
    else:
        (src_r, dst_r, feats, zeros, agg_out,
         src_all, dst_all, rows_v, agg_sh, sem_g) = refs
    c = lax.axis_index("c")
    s = lax.axis_index("s")
    wid = s * NC + c
    r0 = s * ROWS_PER_TILE
    pltpu.sync_copy(zeros.at[pl.ds(r0, ROWS_PER_TILE)],
                    agg_sh.at[pl.ds(r0, ROWS_PER_TILE)])
    if with_deg:
        pltpu.sync_copy(zeros1, deg_v)
    pltpu.sync_copy(src_r.at[wid], src_all)
    pltpu.sync_copy(dst_r.at[wid], dst_all)
    plsc.subcore_barrier()
    ones16 = jnp.full((16,), 1.0, jnp.float32)

    def step(j, carry):
        pltpu.async_copy(feats.at[src_all.at[j]], rows_v, sem_g).wait()
        pltpu.sync_copy(rows_v, agg_sh.at[dst_all.at[j]], add=True)
        if with_deg:
            for k in range(CHUNK // 16):
                plsc.addupdate_scatter(deg_v, [dst_all[j, pl.ds(k * 16, 16)]],
                                       ones16)
        return carry

    lax.fori_loop(0, NCHUNKS, step, 0)
    plsc.subcore_barrier()
    pltpu.sync_copy(agg_sh.at[pl.ds(r0, ROWS_PER_TILE)],
                    agg_out.at[c, pl.ds(r0, ROWS_PER_TILE)])
    if with_deg:
        pltpu.sync_copy(deg_v, deg_out.at[wid])


def _make_sc_layer(with_deg):
    out_type = [jax.ShapeDtypeStruct((NC, N_PAD, D), jnp.float32)]
    scratch = [
        pltpu.VMEM((NCHUNKS, CHUNK), jnp.int32),
        pltpu.VMEM((NCHUNKS, CHUNK), jnp.int32),
        pltpu.VMEM((CHUNK, D), jnp.float32),
    ]
    if with_deg:
        out_type.append(jax.ShapeDtypeStruct((NW, N_PAD), jnp.float32))
        scratch.append(pltpu.VMEM((N_PAD,), jnp.float32))
    scratch.append(pltpu.VMEM_SHARED((N_PAD, D), jnp.float32))
    scratch.append(pltpu.SemaphoreType.DMA)
    return pl.kernel(
        functools.partial(_sc_layer_body, with_deg),
        out_type=tuple(out_type),
        mesh=_MESH,
        scratch_types=scratch,
        compiler_params=pltpu.CompilerParams(needs_layout_passes=False),
    )


_sc_layer1 = _make_sc_layer(True)
_sc_layer2 = _make_sc_layer(False)


def _gather_body(feats, idx, out, idx_v, rows_v, sem):
    c = lax.axis_index("c")
    s = lax.axis_index("s")
    base = (s * NC + c) * B_PER_W
    pltpu.sync_copy(idx.at[pl.ds(base, B_PER_W)], idx_v)
    pltpu.async_copy(feats.at[idx_v], rows_v, sem).wait()
    pltpu.sync_copy(rows_v, out.at[pl.ds(base, B_PER_W)])


_sc_gather = pl.kernel(
    _gather_body,
    out_type=jax.ShapeDtypeStruct((B_OUT, D), jnp.float32),
    mesh=_MESH,
    scratch_types=[
        pltpu.VMEM((B_PER_W,), jnp.int32),
        pltpu.VMEM((B_PER_W, D), jnp.float32),
        pltpu.SemaphoreType.DMA,
    ],
)


def _dense_body(agg_ref, feats_ref, deg_ref, w_ref, b_ref, g_ref, be_ref,
                out_ref):
    d = jnp.sum(deg_ref[...], axis=0)[:, None]
    x = (agg_ref[0] + agg_ref[1] + feats_ref[...]) / (d + 1.0)
    h = jnp.dot(x, w_ref[...], preferred_element_type=jnp.float32) + b_ref[...]
    mu = jnp.mean(h, axis=1, keepdims=True)
    var = jnp.mean((h - mu) * (h - mu), axis=1, keepdims=True)
    h = (h - mu) * lax.rsqrt(var + 1e-5) * g_ref[...] + be_ref[...]
    out_ref[...] = jnp.where(h > 0, h, jnp.exp(h) - 1.0)


_DENSE_BLK = 1024


def _dense(agg, feats, deg, W, b, g, be):
    grid = (N_PAD // _DENSE_BLK,)
    return pl.pallas_call(
        _dense_body,
        grid=grid,
        in_specs=[
            pl.BlockSpec((NC, _DENSE_BLK, D), lambda i: (0, i, 0)),
            pl.BlockSpec((_DENSE_BLK, D), lambda i: (i, 0)),
            pl.BlockSpec((NW, _DENSE_BLK), lambda i: (0, i)),
            pl.BlockSpec((D, D), lambda i: (0, 0)),
            pl.BlockSpec((1, D), lambda i: (0, 0)),
            pl.BlockSpec((1, D), lambda i: (0, 0)),
            pl.BlockSpec((1, D), lambda i: (0, 0)),
        ],
        out_specs=pl.BlockSpec((_DENSE_BLK, D), lambda i: (i, 0)),
        out_shape=jax.ShapeDtypeStruct((N_PAD, D), jnp.float32),
    )(agg, feats, deg, W, b.reshape(1, D), g.reshape(1, D), be.reshape(1, D))


def kernel(embedding, W0, b0, g0, be0, W1, b1, g1, be1, edge_index, index):
    src = edge_index[0].astype(jnp.int32)
    dst = edge_index[1].astype(jnp.int32)
    pad = jnp.full((E_PAD - E,), N, jnp.int32)
    src_r = jnp.concatenate([src, pad]).reshape(NW, NCHUNKS, CHUNK)
    dst_r = jnp.concatenate([dst, pad]).reshape(NW, NCHUNKS, CHUNK)
    feats0 = jnp.pad(embedding, ((0, N_PAD - N), (0, 0)))
    zeros = jnp.zeros((N_PAD, D), jnp.float32)
    zeros1 = jnp.zeros((N_PAD,), jnp.float32)

    agg1, deg = _sc_layer1(src_r, dst_r, feats0, zeros, zeros1)
    f1 = _dense(agg1, feats0, deg, W0, b0, g0, be0)
    agg2, = _sc_layer2(src_r, dst_r, f1, zeros)
    f2 = _dense(agg2, f1, deg, W1, b1, g1, be1)
    return _sc_gather(f2, index.astype(jnp.int32))


# speedup vs baseline: 1.2189x; 1.2189x over previous
import functools

import jax
import jax.numpy as jnp
from jax import lax
from jax.experimental import pallas as pl
from jax.experimental.pallas import tpu as pltpu
from jax.experimental.pallas import tpu_sc as plsc

N = 10000
E = 320000
D = 128
NC, NS = 2, 16
NW = NC * NS
CHUNK = 128
NCHUNKS = -(-E // (NW * CHUNK))
E_PAD = NW * NCHUNKS * CHUNK
N_PAD = 10240
ROWS_PER_TILE = N_PAD // NS
DEG_W = 16
B_OUT = 1024
B_PER_W = B_OUT // NW

_MESH = plsc.VectorSubcoreMesh(core_axis_name="c", subcore_axis_name="s",
                               num_cores=NC, num_subcores=NS)


def _sc_layer_body(with_deg, *refs):
    if with_deg:
        (src_r, dst_r, feats, zeros, zeros1, agg_out, deg_out,
         src_v, dst_v, rows_v, deg_v, agg_sh, sem_g) = refs
    else:
        (src_r, dst_r, feats, zeros, agg_out,
         src_v, dst_v, rows_v, agg_sh, sem_g) = refs
    c = lax.axis_index("c")
    s = lax.axis_index("s")
    wid = s * NC + c
    r0 = s * ROWS_PER_TILE
    pltpu.sync_copy(zeros.at[pl.ds(r0, ROWS_PER_TILE)],
                    agg_sh.at[pl.ds(r0, ROWS_PER_TILE)])
    if with_deg:
        pltpu.sync_copy(zeros1, deg_v)
    plsc.subcore_barrier()
    ones16 = jnp.full((16,), 1.0, jnp.float32)

    def step(j, carry):
        pltpu.sync_copy(src_r.at[wid, j], src_v)
        pltpu.sync_copy(dst_r.at[wid, j], dst_v)
        pltpu.async_copy(feats.at[src_v], rows_v, sem_g).wait()
        pltpu.sync_copy(rows_v, agg_sh.at[dst_v], add=True)
        if with_deg:
            for k in range(CHUNK // 16):
                plsc.addupdate_scatter(deg_v, [dst_v[pl.ds(k * 16, 16)]],
                                       ones16)
        return carry

    lax.fori_loop(0, NCHUNKS, step, 0)
    plsc.subcore_barrier()
    pltpu.sync_copy(agg_sh.at[pl.ds(r0, ROWS_PER_TILE)],
                    agg_out.at[c, pl.ds(r0, ROWS_PER_TILE)])
    if with_deg:
        pltpu.sync_copy(deg_v, deg_out.at[wid])


def _make_sc_layer(with_deg):
    out_type = [jax.ShapeDtypeStruct((NC, N_PAD, D), jnp.float32)]
    scratch = [
        pltpu.VMEM((CHUNK,), jnp.int32),
        pltpu.VMEM((CHUNK,), jnp.int32),
        pltpu.VMEM((CHUNK, D), jnp.float32),
    ]
    if with_deg:
        out_type.append(jax.ShapeDtypeStruct((NW, N_PAD), jnp.float32))
        scratch.append(pltpu.VMEM((N_PAD,), jnp.float32))
    scratch.append(pltpu.VMEM_SHARED((N_PAD, D), jnp.float32))
    scratch.append(pltpu.SemaphoreType.DMA)
    return pl.kernel(
        functools.partial(_sc_layer_body, with_deg),
        out_type=tuple(out_type),
        mesh=_MESH,
        scratch_types=scratch,
        compiler_params=pltpu.CompilerParams(needs_layout_passes=False),
    )


_sc_layer1 = _make_sc_layer(True)
_sc_layer2 = _make_sc_layer(False)


def _gather_body(feats, idx, out, idx_v, rows_v, sem):
    c = lax.axis_index("c")
    s = lax.axis_index("s")
    base = (s * NC + c) * B_PER_W
    pltpu.sync_copy(idx.at[pl.ds(base, B_PER_W)], idx_v)
    pltpu.async_copy(feats.at[idx_v], rows_v, sem).wait()
    pltpu.sync_copy(rows_v, out.at[pl.ds(base, B_PER_W)])


_sc_gather = pl.kernel(
    _gather_body,
    out_type=jax.ShapeDtypeStruct((B_OUT, D), jnp.float32),
    mesh=_MESH,
    scratch_types=[
        pltpu.VMEM((B_PER_W,), jnp.int32),
        pltpu.VMEM((B_PER_W, D), jnp.float32),
        pltpu.SemaphoreType.DMA,
    ],
)


def _dense_body(agg_ref, feats_ref, deg_ref, w_ref, b_ref, g_ref, be_ref,
                out_ref):
    d = jnp.sum(deg_ref[...], axis=0)[:, None]
    x = (agg_ref[0] + agg_ref[1] + feats_ref[...]) / (d + 1.0)
    h = jnp.dot(x, w_ref[...], preferred_element_type=jnp.float32) + b_ref[...]
    mu = jnp.mean(h, axis=1, keepdims=True)
    var = jnp.mean((h - mu) * (h - mu), axis=1, keepdims=True)
    h = (h - mu) * lax.rsqrt(var + 1e-5) * g_ref[...] + be_ref[...]
    out_ref[...] = jnp.where(h > 0, h, jnp.exp(h) - 1.0)


_DENSE_BLK = 1024


def _dense(agg, feats, deg, W, b, g, be):
    grid = (N_PAD // _DENSE_BLK,)
    return pl.pallas_call(
        _dense_body,
        grid=grid,
        in_specs=[
            pl.BlockSpec((NC, _DENSE_BLK, D), lambda i: (0, i, 0)),
            pl.BlockSpec((_DENSE_BLK, D), lambda i: (i, 0)),
            pl.BlockSpec((NW, _DENSE_BLK), lambda i: (0, i)),
            pl.BlockSpec((D, D), lambda i: (0, 0)),
            pl.BlockSpec((1, D), lambda i: (0, 0)),
            pl.BlockSpec((1, D), lambda i: (0, 0)),
            pl.BlockSpec((1, D), lambda i: (0, 0)),
        ],
        out_specs=pl.BlockSpec((_DENSE_BLK, D), lambda i: (i, 0)),
        out_shape=jax.ShapeDtypeStruct((N_PAD, D), jnp.float32),
    )(agg, feats, deg, W, b.reshape(1, D), g.reshape(1, D), be.reshape(1, D))


def kernel(embedding, W0, b0, g0, be0, W1, b1, g1, be1, edge_index, index):
    src = edge_index[0].astype(jnp.int32)
    dst = edge_index[1].astype(jnp.int32)
    pad = jnp.full((E_PAD - E,), N, jnp.int32)
    src_r = jnp.concatenate([src, pad]).reshape(NW, NCHUNKS, CHUNK)
    dst_r = jnp.concatenate([dst, pad]).reshape(NW, NCHUNKS, CHUNK)
    feats0 = jnp.pad(embedding, ((0, N_PAD - N), (0, 0)))
    zeros = jnp.zeros((N_PAD, D), jnp.float32)
    zeros1 = jnp.zeros((N_PAD,), jnp.float32)

    agg1, deg = _sc_layer1(src_r, dst_r, feats0, zeros, zeros1)
    f1 = _dense(agg1, feats0, deg, W0, b0, g0, be0)
    agg2, = _sc_layer2(src_r, dst_r, f1, zeros)
    f2 = _dense(agg2, f1, deg, W1, b1, g1, be1)
    return _sc_gather(f2, index.astype(jnp.int32))


# speedup vs baseline: 1.2192x; 1.0003x over previous
import functools

import jax
import jax.numpy as jnp
from jax import lax
from jax.experimental import pallas as pl
from jax.experimental.pallas import tpu as pltpu
from jax.experimental.pallas import tpu_sc as plsc

N = 10000
E = 320000
D = 128
NC, NS = 2, 16
NW = NC * NS
CHUNK = 128
NCHUNKS = -(-E // (NW * CHUNK))
E_PAD = NW * NCHUNKS * CHUNK
N_PAD = 10240
ROWS_PER_TILE = N_PAD // NS
B_OUT = 1024
B_PER_W = B_OUT // NW

_MESH = plsc.VectorSubcoreMesh(core_axis_name="c", subcore_axis_name="s",
                               num_cores=NC, num_subcores=NS)


def _sc_layer_body(with_deg, *refs):
    if with_deg:
        (src_r, dst_r, feats, zeros, zeros1, agg_out, deg_out,
         src_v, dst_v, rows_v, deg_v, agg_sh, sem_g) = refs
    else:
        (src_r, dst_r, feats, zeros, agg_out,
         src_v, dst_v, rows_v, agg_sh, sem_g) = refs
    c = lax.axis_index("c")
    s = lax.axis_index("s")
    wid = s * NC + c
    r0 = s * ROWS_PER_TILE
    pltpu.sync_copy(zeros.at[pl.ds(r0, ROWS_PER_TILE)],
                    agg_sh.at[pl.ds(r0, ROWS_PER_TILE)])
    if with_deg:
        pltpu.sync_copy(zeros1, deg_v)
    plsc.subcore_barrier()
    ones16 = jnp.full((16,), 1.0, jnp.float32)

    def step(j, carry):
        pltpu.sync_copy(src_r.at[wid, j], src_v)
        pltpu.sync_copy(dst_r.at[wid, j], dst_v)
        pltpu.async_copy(feats.at[src_v], rows_v, sem_g).wait()
        pltpu.sync_copy(rows_v, agg_sh.at[dst_v], add=True)
        if with_deg:
            for k in range(CHUNK // 16):
                plsc.addupdate_scatter(deg_v, [dst_v[pl.ds(k * 16, 16)]],
                                       ones16)
        return carry

    lax.fori_loop(0, NCHUNKS, step, 0)
    plsc.subcore_barrier()
    pltpu.sync_copy(agg_sh.at[pl.ds(r0, ROWS_PER_TILE)],
                    agg_out.at[c, pl.ds(r0, ROWS_PER_TILE)])
    if with_deg:
        pltpu.sync_copy(deg_v, deg_out.at[wid])


def _make_sc_layer(with_deg):
    out_type = [jax.ShapeDtypeStruct((NC, N_PAD, D), jnp.float32)]
    scratch = [
        pltpu.VMEM((CHUNK,), jnp.int32),
        pltpu.VMEM((CHUNK,), jnp.int32),
        pltpu.VMEM((CHUNK, D), jnp.float32),
    ]
    if with_deg:
        out_type.append(jax.ShapeDtypeStruct((NW, N_PAD), jnp.float32))
        scratch.append(pltpu.VMEM((N_PAD,), jnp.float32))
    scratch.append(pltpu.VMEM_SHARED((N_PAD, D), jnp.float32))
    scratch.append(pltpu.SemaphoreType.DMA)
    return pl.kernel(
        functools.partial(_sc_layer_body, with_deg),
        out_type=tuple(out_type),
        mesh=_MESH,
        scratch_types=scratch,
        compiler_params=pltpu.CompilerParams(needs_layout_passes=False),
    )


_sc_layer1 = _make_sc_layer(True)
_sc_layer2 = _make_sc_layer(False)


def _gather_body(feats, idx, out, idx_v, rows_v, sem):
    c = lax.axis_index("c")
    s = lax.axis_index("s")
    base = (s * NC + c) * B_PER_W
    pltpu.sync_copy(idx.at[pl.ds(base, B_PER_W)], idx_v)
    pltpu.async_copy(feats.at[idx_v], rows_v, sem).wait()
    pltpu.sync_copy(rows_v, out.at[pl.ds(base, B_PER_W)])


_sc_gather = pl.kernel(
    _gather_body,
    out_type=jax.ShapeDtypeStruct((B_OUT, D), jnp.float32),
    mesh=_MESH,
    scratch_types=[
        pltpu.VMEM((B_PER_W,), jnp.int32),
        pltpu.VMEM((B_PER_W, D), jnp.float32),
        pltpu.SemaphoreType.DMA,
    ],
)


def _dense_body(agg_ref, feats_ref, deg_ref, w_ref, b_ref, g_ref, be_ref,
                out_ref):
    d = jnp.sum(deg_ref[...], axis=0)[:, None]
    x = (agg_ref[0] + agg_ref[1] + feats_ref[...]) / (d + 1.0)
    h = jnp.dot(x, w_ref[...], preferred_element_type=jnp.float32) + b_ref[...]
    mu = jnp.mean(h, axis=1, keepdims=True)
    var = jnp.mean((h - mu) * (h - mu), axis=1, keepdims=True)
    h = (h - mu) * lax.rsqrt(var + 1e-5) * g_ref[...] + be_ref[...]
    out_ref[...] = jnp.where(h > 0, h, jnp.exp(h) - 1.0)


_DENSE_BLK = 1024


def _dense(agg, feats, deg, W, b, g, be):
    grid = (N_PAD // _DENSE_BLK,)
    return pl.pallas_call(
        _dense_body,
        grid=grid,
        in_specs=[
            pl.BlockSpec((NC, _DENSE_BLK, D), lambda i: (0, i, 0)),
            pl.BlockSpec((_DENSE_BLK, D), lambda i: (i, 0)),
            pl.BlockSpec((NW, _DENSE_BLK), lambda i: (0, i)),
            pl.BlockSpec((D, D), lambda i: (0, 0)),
            pl.BlockSpec((1, D), lambda i: (0, 0)),
            pl.BlockSpec((1, D), lambda i: (0, 0)),
            pl.BlockSpec((1, D), lambda i: (0, 0)),
        ],
        out_specs=pl.BlockSpec((_DENSE_BLK, D), lambda i: (i, 0)),
        out_shape=jax.ShapeDtypeStruct((N_PAD, D), jnp.float32),
    )(agg, feats, deg, W, b.reshape(1, D), g.reshape(1, D), be.reshape(1, D))


def kernel(embedding, W0, b0, g0, be0, W1, b1, g1, be1, edge_index, index):
    src = edge_index[0].astype(jnp.int32)
    dst = edge_index[1].astype(jnp.int32)
    pad = jnp.full((E_PAD - E,), N, jnp.int32)
    src_r = jnp.concatenate([src, pad]).reshape(NW, NCHUNKS, CHUNK)
    dst_r = jnp.concatenate([dst, pad]).reshape(NW, NCHUNKS, CHUNK)
    feats0 = jnp.pad(embedding, ((0, N_PAD - N), (0, 0)))
    zeros = jnp.zeros((N_PAD, D), jnp.float32)
    zeros1 = jnp.zeros((N_PAD,), jnp.float32)

    agg1, deg = _sc_layer1(src_r, dst_r, feats0, zeros, zeros1)
    f1 = _dense(agg1, feats0, deg, W0, b0, g0, be0)
    agg2, = _sc_layer2(src_r, dst_r, f1, zeros)
    f2 = _dense(agg2, f1, deg, W1, b1, g1, be1)
    return _sc_gather(f2, index.astype(jnp.int32))
